# Initial kernel scaffold; baseline (speedup 1.0000x reference)
import jax
import jax.numpy as jnp
from jax.experimental import pallas as pl


def kernel(x, edge_index, batch, W1, b1, W2, b2, W3, b3, W4, b4, We1, be1, We2, be2, Wd, bd):
    raise NotImplementedError("write your pallas kernel here")


# speedup vs baseline: 3.7187x; 3.7187x over previous
import functools

import jax
import jax.numpy as jnp
from jax import lax
from jax.experimental import pallas as pl
from jax.experimental.pallas import tpu as pltpu
from jax.experimental.pallas import tpu_sc as plsc

N = 10000
E = 320000
DI = 128
H = 64
NG = 64
ND = 3

NC = 2
NS = 16
NW = NC * NS

C = 80
CPW = E // C // NW
EPW = E // NW
ZCH = 640
ZLAST = N - ZCH * (NS - 1)

_f32 = jnp.float32

_SC_PARAMS = pltpu.CompilerParams(use_tc_tiling_on_sc=False)


def _mesh():
    return plsc.VectorSubcoreMesh(
        core_axis_name="c", subcore_axis_name="s",
        num_cores=NC, num_subcores=NS,
    )


@functools.partial(
    pl.kernel,
    out_type=jax.ShapeDtypeStruct((NC, N, H), _f32),
    mesh=_mesh(),
    compiler_params=_SC_PARAMS,
    scratch_types=[
        pltpu.VMEM((CPW, C), jnp.int32),
        pltpu.VMEM((CPW, C), jnp.int32),
        pltpu.VMEM((C, H), _f32),
        pltpu.VMEM_SHARED((N, H), _f32),
        pltpu.SemaphoreType.DMA,
    ],
)
def _segsum_sc(table_h, src_h, dst_h, zeros_h, out_h, src_v, dst_v, rows_v,
               acc_sh, gsem):
    c = lax.axis_index("c")
    s = lax.axis_index("s")
    wid = s * NC + c
    pltpu.sync_copy(src_h.at[wid], src_v)
    pltpu.sync_copy(dst_h.at[wid], dst_v)
    z0 = s * ZCH
    zn = jnp.where(s == NS - 1, ZLAST, ZCH)
    pltpu.sync_copy(zeros_h.at[pl.ds(0, zn)], acc_sh.at[pl.ds(z0, zn)])
    plsc.subcore_barrier()

    @pl.loop(0, CPW)
    def _(j):
        pltpu.async_copy(table_h.at[src_v.at[j]], rows_v, gsem).wait()
        pltpu.sync_copy(rows_v, acc_sh.at[dst_v.at[j]], add=True)

    plsc.subcore_barrier()
    pltpu.sync_copy(acc_sh.at[pl.ds(z0, zn)], out_h.at[c, pl.ds(z0, zn)])


@functools.partial(
    pl.kernel,
    out_type=(
        jax.ShapeDtypeStruct((E, H), _f32),
        jax.ShapeDtypeStruct((E, H), _f32),
    ),
    mesh=_mesh(),
    compiler_params=_SC_PARAMS,
    scratch_types=[
        pltpu.VMEM((CPW, C), jnp.int32),
        pltpu.VMEM((CPW, C), jnp.int32),
        pltpu.VMEM((C, H), _f32),
        pltpu.VMEM((C, H), _f32),
        pltpu.SemaphoreType.DMA,
        pltpu.SemaphoreType.DMA,
    ],
)
def _pair_gather_sc(table_h, src_h, dst_h, souts_h, doutd_h, src_v, dst_v,
                    sbuf, dbuf, sem1, sem2):
    c = lax.axis_index("c")
    s = lax.axis_index("s")
    wid = s * NC + c
    e0 = wid * EPW
    pltpu.sync_copy(src_h.at[wid], src_v)
    pltpu.sync_copy(dst_h.at[wid], dst_v)

    @pl.loop(0, CPW)
    def _(j):
        cp1 = pltpu.async_copy(table_h.at[src_v.at[j]], sbuf, sem1)
        cp2 = pltpu.async_copy(table_h.at[dst_v.at[j]], dbuf, sem2)
        cp1.wait()
        pltpu.sync_copy(sbuf, souts_h.at[pl.ds(e0 + j * C, C)])
        cp2.wait()
        pltpu.sync_copy(dbuf, doutd_h.at[pl.ds(e0 + j * C, C)])


@functools.partial(
    pl.kernel,
    out_type=jax.ShapeDtypeStruct((NC, N), _f32),
    mesh=_mesh(),
    compiler_params=_SC_PARAMS,
    scratch_types=[
        pltpu.VMEM((CPW, C), jnp.int32),
        pltpu.VMEM((CPW, C), jnp.int32),
        pltpu.VMEM((CPW, C), _f32),
        pltpu.VMEM_SHARED((N,), _f32),
    ],
)
def _scatter_w_sc(es_h, src_h, dst_h, zeros_h, out_h, src_v, dst_v, es_v,
                  w_sh):
    c = lax.axis_index("c")
    s = lax.axis_index("s")
    wid = s * NC + c
    pltpu.sync_copy(src_h.at[wid], src_v)
    pltpu.sync_copy(dst_h.at[wid], dst_v)
    pltpu.sync_copy(es_h.at[wid], es_v)

    @pl.when(s == 0)
    def _():
        pltpu.sync_copy(zeros_h, w_sh)

    plsc.subcore_barrier()

    @pl.loop(0, CPW)
    def _(j):
        pltpu.sync_copy(es_v.at[j], w_sh.at[src_v.at[j]], add=True)
        pltpu.sync_copy(es_v.at[j], w_sh.at[dst_v.at[j]], add=True)

    plsc.subcore_barrier()

    @pl.when(s == 0)
    def _():
        pltpu.sync_copy(w_sh, out_h.at[c])


NB = 1000


def _dot(a, b):
    return jnp.dot(a, b, preferred_element_type=_f32)


def _mm_body(x_ref, w_ref, o_ref):
    o_ref[...] = _dot(x_ref[...], w_ref[...])


def _proj(x, W):
    K = x.shape[1]
    return pl.pallas_call(
        _mm_body,
        grid=(N // NB,),
        in_specs=[
            pl.BlockSpec((NB, K), lambda i: (i, 0)),
            pl.BlockSpec((K, H), lambda i: (0, 0)),
        ],
        out_specs=pl.BlockSpec((NB, H), lambda i: (i, 0)),
        out_shape=jax.ShapeDtypeStruct((N, H), _f32),
    )(x, W)


def _gin_tail_body(y_ref, a0_ref, a1_ref, ba_ref, wb_ref, bb_ref, o_ref):
    z = y_ref[...] + a0_ref[...] + a1_ref[...] + ba_ref[...]
    t = _dot(jnp.maximum(z, 0.0), wb_ref[...]) + bb_ref[...]
    o_ref[...] = jnp.maximum(t, 0.0)


def _gin_fused_body(y_ref, a0_ref, a1_ref, ba_ref, wb_ref, bb_ref, wn_ref,
                    o_ref):
    z = y_ref[...] + a0_ref[...] + a1_ref[...] + ba_ref[...]
    t = _dot(jnp.maximum(z, 0.0), wb_ref[...]) + bb_ref[...]
    o_ref[...] = _dot(jnp.maximum(t, 0.0), wn_ref[...])


def _gin_stage(y, a, ba, Wb, bb, Wn):
    base_specs = [
        pl.BlockSpec((NB, H), lambda i: (i, 0)),
        pl.BlockSpec((NB, H), lambda i: (i, 0)),
        pl.BlockSpec((NB, H), lambda i: (i, 0)),
        pl.BlockSpec((1, H), lambda i: (0, 0)),
        pl.BlockSpec((H, H), lambda i: (0, 0)),
        pl.BlockSpec((1, H), lambda i: (0, 0)),
    ]
    args = [y, a[0], a[1], ba.reshape(1, H), Wb, bb.reshape(1, H)]
    if Wn is None:
        body = _gin_tail_body
        specs = base_specs
    else:
        body = _gin_fused_body
        specs = base_specs + [pl.BlockSpec((H, H), lambda i: (0, 0))]
        args = args + [Wn]
    return pl.pallas_call(
        body,
        grid=(N // NB,),
        in_specs=specs,
        out_specs=pl.BlockSpec((NB, H), lambda i: (i, 0)),
        out_shape=jax.ShapeDtypeStruct((N, H), _f32),
    )(*args)


EB = 512


def _edge_head_body(s_ref, d_ref, wa_ref, wb_ref, wc_ref, be1_ref, we2_ref,
                    be2_ref, o_ref):
    sv = s_ref[...]
    dv = d_ref[...]
    u = (_dot(sv, wa_ref[...]) + _dot(dv, wb_ref[...])
         + _dot(jnp.abs(sv - dv), wc_ref[...]) + be1_ref[...])
    v = _dot(jnp.maximum(u, 0.0), we2_ref[...]) + be2_ref[...]
    es = 1.0 / (1.0 + jnp.exp(-v))
    o_ref[...] = jnp.reshape(es, (EB,))


def _edge_head(sg, dg, We1, be1, We2, be2):
    wa = We1[0:H]
    wb = We1[H:2 * H]
    wc = We1[2 * H:3 * H]
    return pl.pallas_call(
        _edge_head_body,
        grid=(E // EB,),
        in_specs=[
            pl.BlockSpec((EB, H), lambda i: (i, 0)),
            pl.BlockSpec((EB, H), lambda i: (i, 0)),
            pl.BlockSpec((H, H), lambda i: (0, 0)),
            pl.BlockSpec((H, H), lambda i: (0, 0)),
            pl.BlockSpec((H, H), lambda i: (0, 0)),
            pl.BlockSpec((1, H), lambda i: (0, 0)),
            pl.BlockSpec((H, 1), lambda i: (0, 0)),
            pl.BlockSpec((1, 1), lambda i: (0, 0)),
        ],
        out_specs=pl.BlockSpec((EB,), lambda i: (i,)),
        out_shape=jax.ShapeDtypeStruct((E,), _f32),
    )(sg, dg, wa, wb, wc, be1.reshape(1, H), We2, be2.reshape(1, 1))


def _pool_body(h_ref, w0_ref, w1_ref, batch_ref, wd_ref, bd_ref, ge_ref,
               lg_ref, pooled_acc, norm_acc):
    i = pl.program_id(0)

    @pl.when(i == 0)
    def _():
        pooled_acc[...] = jnp.zeros_like(pooled_acc)
        norm_acc[...] = jnp.zeros_like(norm_acc)

    wn = w0_ref[...] + w1_ref[...] + 1e-6
    bi = batch_ref[0]
    gids = lax.broadcasted_iota(jnp.int32, (NG, NB), 0)
    onehot = (gids == bi).astype(_f32)
    hw = h_ref[...] * wn
    pooled_acc[...] += _dot(onehot, hw)
    norm_acc[...] += _dot(onehot, wn)

    @pl.when(i == pl.num_programs(0) - 1)
    def _():
        norm = jnp.maximum(norm_acc[...], 1e-6)
        ge = pooled_acc[...] / norm
        ge_ref[...] = ge
        lg_ref[...] = _dot(ge, wd_ref[...]) + bd_ref[...]


def _pool(h, w0, w1, batch3, Wd, bd):
    return pl.pallas_call(
        _pool_body,
        grid=(N // NB,),
        in_specs=[
            pl.BlockSpec((NB, H), lambda i: (i, 0)),
            pl.BlockSpec((NB, 1), lambda i: (i, 0)),
            pl.BlockSpec((NB, 1), lambda i: (i, 0)),
            pl.BlockSpec((1, 1, NB), lambda i: (i, 0, 0)),
            pl.BlockSpec((H, ND), lambda i: (0, 0)),
            pl.BlockSpec((1, ND), lambda i: (0, 0)),
        ],
        out_specs=(
            pl.BlockSpec((NG, H), lambda i: (0, 0)),
            pl.BlockSpec((NG, ND), lambda i: (0, 0)),
        ),
        out_shape=(
            jax.ShapeDtypeStruct((NG, H), _f32),
            jax.ShapeDtypeStruct((NG, ND), _f32),
        ),
        scratch_shapes=[
            pltpu.VMEM((NG, H), _f32),
            pltpu.VMEM((NG, 1), _f32),
        ],
    )(h, w0, w1, batch3, Wd, bd)


def kernel(x, edge_index, batch, W1, b1, W2, b2, W3, b3, W4, b4, We1, be1,
           We2, be2, Wd, bd):
    src3 = edge_index[0].reshape(NW, CPW, C)
    dst3 = edge_index[1].reshape(NW, CPW, C)
    zeros_h = jnp.zeros((ZCH, H), _f32)
    zeros_w = jnp.zeros((N,), _f32)

    y1 = _proj(x, W1)
    a1 = _segsum_sc(y1, src3, dst3, zeros_h)
    y2 = _gin_stage(y1, a1, b1, W2, b2, W3)
    a2 = _segsum_sc(y2, src3, dst3, zeros_h)
    h = _gin_stage(y2, a2, b3, W4, b4, None)

    sg, dg = _pair_gather_sc(h, src3, dst3)
    edge_scores = _edge_head(sg, dg, We1, be1, We2, be2)

    es3 = edge_scores.reshape(NW, CPW, C)
    wparts = _scatter_w_sc(es3, src3, dst3, zeros_w)
    w0 = wparts[0].reshape(N, 1)
    w1 = wparts[1].reshape(N, 1)

    ge, logits = _pool(h, w0, w1, batch.reshape(N // NB, 1, NB), Wd,
                       bd.reshape(1, ND))
    return edge_scores, logits, h, ge
